# baseline (speedup 1.0000x reference)
import functools

import jax
import jax.numpy as jnp
from jax import lax
from jax.experimental import pallas as pl
from jax.experimental.pallas import tpu as pltpu
from jax.experimental.pallas import tpu_sc as plsc

N = 10000
E = 320000
D = 128
OUT = 16
K = 4

NC = 2
NS = 16
NW = NC * NS
EPW = E // NW
CHUNK = 80
NCHUNK = EPW // CHUNK
ACC_ROWS = 10240

R = 400
NBLK = N // R

def _build_gather_sum():
    mesh = plsc.VectorSubcoreMesh(core_axis_name="c", subcore_axis_name="s")

    @functools.partial(
        pl.kernel,
        out_type=jax.ShapeDtypeStruct((NC, ACC_ROWS, D), jnp.float32),
        mesh=mesh,
        scratch_types=[
            pltpu.VMEM((EPW,), jnp.int32),
            pltpu.VMEM((NCHUNK, CHUNK), jnp.int32),
            pltpu.VMEM((2 * CHUNK, D), jnp.float32),
            pltpu.VMEM_SHARED((ACC_ROWS, D), jnp.float32),
            pltpu.SemaphoreType.DMA,
            pltpu.SemaphoreType.DMA,
            pltpu.SemaphoreType.DMA,
        ],
    )
    def gs(z_hbm, src_hbm, dst_hbm, out_hbm, sidx, didx, rows, acc, gsem, isem, ssem):
        c = lax.axis_index("c")
        s = lax.axis_index("s")
        wid = c * NS + s

        cp_s = pltpu.async_copy(src_hbm.at[wid], sidx, isem)
        cp_d = pltpu.async_copy(dst_hbm.at[wid], didx, isem)

        zeros = jnp.zeros((16,), jnp.float32)
        for i in range(16):
            for j in range(D // 16):
                rows[i, pl.ds(16 * j, 16)] = zeros

        @pl.loop(0, 640 // 16)
        def _zero(k):
            pltpu.sync_copy(rows.at[pl.ds(0, 16)], acc.at[pl.ds(s * 640 + k * 16, 16)])

        cp_s.wait()
        cp_d.wait()
        plsc.subcore_barrier()

        pltpu.async_copy(z_hbm.at[sidx.at[pl.ds(0, CHUNK)]], rows.at[pl.ds(0, CHUNK)], gsem)

        @pl.loop(0, NCHUNK)
        def _edges(k):
            b = (k % 2) * CHUNK
            pltpu.make_async_copy(
                z_hbm.at[sidx.at[pl.ds(k * CHUNK, CHUNK)]], rows.at[pl.ds(b, CHUNK)], gsem
            ).wait()

            @pl.when(k + 1 < NCHUNK)
            def _():
                nb = ((k + 1) % 2) * CHUNK

                @pl.when(k >= 1)
                def _():
                    pltpu.make_async_copy(
                        z_hbm.at[pl.ds(0, CHUNK)], rows.at[pl.ds(nb, CHUNK)], ssem
                    ).wait()

                pltpu.async_copy(
                    z_hbm.at[sidx.at[pl.ds((k + 1) * CHUNK, CHUNK)]],
                    rows.at[pl.ds(nb, CHUNK)], gsem
                )

            pltpu.async_copy(rows.at[pl.ds(b, CHUNK)], acc.at[didx.at[k]], ssem, add=True)

        pltpu.make_async_copy(z_hbm.at[pl.ds(0, 2 * CHUNK)], rows, ssem).wait()
        plsc.subcore_barrier()
        pltpu.sync_copy(acc.at[pl.ds(s * 640, 640)], out_hbm.at[c, pl.ds(s * 640, 640)])

    return gs


_gather_sum_cache = []


def _gather_sum(z, src, dst):
    if not _gather_sum_cache:
        _gather_sum_cache.append(_build_gather_sum())
    return _gather_sum_cache[0](z, src, dst)


def _combine_stats_body(h_ref, g1_ref, g2_ref, g3_ref, dis_ref, V_ref, b_ref,
                        S_ref, st_ref, acc_ref):
    i = pl.program_id(0)
    d = dis_ref[...]
    u1 = d * (g1_ref[0] + g1_ref[1])
    u2 = d * (g2_ref[0] + g2_ref[1])
    u3 = d * (g3_ref[0] + g3_ref[1])
    S = jnp.dot(h_ref[...], V_ref[0], preferred_element_type=jnp.float32)
    S += jnp.dot(u1, V_ref[1], preferred_element_type=jnp.float32)
    S += jnp.dot(u2, V_ref[2], preferred_element_type=jnp.float32)
    S += jnp.dot(u3, V_ref[3], preferred_element_type=jnp.float32)
    S += b_ref[...]
    S_ref[...] = S

    @pl.when(i == 0)
    def _():
        acc_ref[...] = jnp.zeros_like(acc_ref)

    acc_ref[0:1, :] += jnp.sum(S, axis=0, keepdims=True)
    acc_ref[1:2, :] += jnp.sum(S * S, axis=0, keepdims=True)

    @pl.when(i == NBLK - 1)
    def _():
        st_ref[...] = acc_ref[...]


def _combine_stats(h, g1, g2, g3, dis, V, b):
    return pl.pallas_call(
        _combine_stats_body,
        grid=(NBLK,),
        in_specs=[
            pl.BlockSpec((R, D), lambda i: (i, 0)),
            pl.BlockSpec((2, R, D), lambda i: (0, i, 0)),
            pl.BlockSpec((2, R, D), lambda i: (0, i, 0)),
            pl.BlockSpec((2, R, D), lambda i: (0, i, 0)),
            pl.BlockSpec((R, 1), lambda i: (i, 0)),
            pl.BlockSpec((4, D, D), lambda i: (0, 0, 0)),
            pl.BlockSpec((1, D), lambda i: (0, 0)),
        ],
        out_specs=[
            pl.BlockSpec((R, D), lambda i: (i, 0)),
            pl.BlockSpec((8, 128), lambda i: (0, 0)),
        ],
        out_shape=[
            jax.ShapeDtypeStruct((N, D), jnp.float32),
            jax.ShapeDtypeStruct((8, 128), jnp.float32),
        ],
        scratch_shapes=[pltpu.VMEM((8, 128), jnp.float32)],
    )(h, g1, g2, g3, dis, V, b)


def _norm_act_body(S_ref, sc_ref, sh_ref, dis_ref, h_ref, z_ref):
    v = S_ref[...] * sc_ref[...] + sh_ref[...]
    h = jnp.where(v > 0, v, 0.1 * v)
    h_ref[...] = h
    z_ref[...] = dis_ref[...] * h


def _norm_act(S, scale, shift, dis):
    return pl.pallas_call(
        _norm_act_body,
        grid=(NBLK,),
        in_specs=[
            pl.BlockSpec((R, D), lambda i: (i, 0)),
            pl.BlockSpec((1, D), lambda i: (0, 0)),
            pl.BlockSpec((1, D), lambda i: (0, 0)),
            pl.BlockSpec((R, 1), lambda i: (i, 0)),
        ],
        out_specs=[
            pl.BlockSpec((R, D), lambda i: (i, 0)),
            pl.BlockSpec((R, D), lambda i: (i, 0)),
        ],
        out_shape=[
            jax.ShapeDtypeStruct((N, D), jnp.float32),
            jax.ShapeDtypeStruct((N, D), jnp.float32),
        ],
    )(S, scale, shift, dis)


def _norm_res_pool_body(S_ref, sc_ref, sh_ref, x0_ref, st_ref, acc_ref):
    i = pl.program_id(0)
    v = x0_ref[...] + S_ref[...] * sc_ref[...] + sh_ref[...]
    h = jnp.maximum(v, 0.0)

    @pl.when(i == 0)
    def _():
        acc_ref[...] = jnp.zeros_like(acc_ref)

    acc_ref[0:1, :] += jnp.sum(h, axis=0, keepdims=True)

    @pl.when(i == NBLK - 1)
    def _():
        st_ref[...] = acc_ref[...]


def _norm_res_pool(S, scale, shift, x0):
    return pl.pallas_call(
        _norm_res_pool_body,
        grid=(NBLK,),
        in_specs=[
            pl.BlockSpec((R, D), lambda i: (i, 0)),
            pl.BlockSpec((1, D), lambda i: (0, 0)),
            pl.BlockSpec((1, D), lambda i: (0, 0)),
            pl.BlockSpec((R, D), lambda i: (i, 0)),
        ],
        out_specs=pl.BlockSpec((8, 128), lambda i: (0, 0)),
        out_shape=jax.ShapeDtypeStruct((8, 128), jnp.float32),
        scratch_shapes=[pltpu.VMEM((8, 128), jnp.float32)],
    )(S, scale, shift, x0)


def _merge_body(g_ref, d_ref, t_ref, *, a):
    dd = d_ref[...]
    t_ref[...] = a * dd * dd * (g_ref[0] + g_ref[1])


def _merge_aux_body(g_ref, d_ref, aux_ref, t_ref, *, a, c):
    dd = d_ref[...]
    t_ref[...] = a * dd * dd * (g_ref[0] + g_ref[1]) + c * aux_ref[...]


def _merge_t(g, dis, a, aux=None, c=0.0):
    gspec = pl.BlockSpec((2, R, D), lambda i: (0, i, 0))
    dspec = pl.BlockSpec((R, 1), lambda i: (i, 0))
    xspec = pl.BlockSpec((R, D), lambda i: (i, 0))
    oshape = jax.ShapeDtypeStruct((N, D), jnp.float32)
    if aux is None:
        return pl.pallas_call(
            functools.partial(_merge_body, a=a),
            grid=(NBLK,), in_specs=[gspec, dspec], out_specs=xspec,
            out_shape=oshape,
        )(g, dis)
    return pl.pallas_call(
        functools.partial(_merge_aux_body, a=a, c=c),
        grid=(NBLK,), in_specs=[gspec, dspec, xspec], out_specs=xspec,
        out_shape=oshape,
    )(g, dis, aux)


def _scale_body(x_ref, d_ref, z_ref):
    z_ref[...] = d_ref[...] * x_ref[...]


def _scale_rows(x, dis):
    return pl.pallas_call(
        _scale_body,
        grid=(NBLK,),
        in_specs=[
            pl.BlockSpec((R, D), lambda i: (i, 0)),
            pl.BlockSpec((R, 1), lambda i: (i, 0)),
        ],
        out_specs=pl.BlockSpec((R, D), lambda i: (i, 0)),
        out_shape=jax.ShapeDtypeStruct((N, D), jnp.float32),
    )(x, dis)


def kernel(edge_index, feat, feat_1, feat_2,
           W1, b1, gn1_w, gn1_b, gn1_ms, lin1_W, lin1_b,
           W2, b2, gn2_w, gn2_b, gn2_ms, lin2_W, lin2_b,
           W3, b3, gn3_w, gn3_b, gn3_ms, lin3_W, lin3_b):
    src = edge_index[0].astype(jnp.int32)
    dst = edge_index[1].astype(jnp.int32)
    order = jnp.argsort(dst)
    srcp = jnp.take(src, order)
    dstp = jnp.take(dst, order)
    src3 = srcp.reshape(NW, EPW)
    dst3 = dstp.reshape(NW, NCHUNK, CHUNK)
    deg = jax.ops.segment_sum(jnp.ones((E,), jnp.float32), src, num_segments=N)
    dis = jnp.where(deg > 0, 1.0 / jnp.sqrt(jnp.maximum(deg, 1e-12)), 0.0)
    disc = dis[:, None]

    def fold_W(W):
        return jnp.stack([W[0] - W[2], W[3] - W[1], -2.0 * W[2], -2.0 * W[3]])

    def norm_params(st, gw, gb, gms):
        mean = st[0] / N
        ex2 = st[1] / N
        var = ex2 - mean * mean * gms * (2.0 - gms)
        scale = gw / jnp.sqrt(var + 1e-5)
        shift = gb - scale * gms * mean
        return scale[None, :], shift[None, :]

    x0s = [feat, feat_1, feat_2]
    Ws = [W1, W2, W3]
    bs = [b1, b2, b3]
    gws = [gn1_w, gn2_w, gn3_w]
    gbs = [gn1_b, gn2_b, gn3_b]
    gmss = [gn1_ms, gn2_ms, gn3_ms]

    hs = list(x0s)
    zs = [_scale_rows(x, disc) for x in x0s]

    for i in range(4):
        g1s = [_gather_sum(z, src3, dst3) for z in zs]
        t1s = [_merge_t(g1, disc, -1.0) for g1 in g1s]
        g2s = [_gather_sum(t1, src3, dst3) for t1 in t1s]
        t2s = [_merge_t(g2, disc, -2.0, aux=z, c=-1.0)
               for g2, z in zip(g2s, zs)]
        g3s = [_gather_sum(t2, src3, dst3) for t2 in t2s]
        Ss, sts = [], []
        for b_i in range(3):
            S, st = _combine_stats(hs[b_i], g1s[b_i], g2s[b_i], g3s[b_i],
                                   disc, fold_W(Ws[b_i][i]), bs[b_i][i][None, :])
            Ss.append(S)
            sts.append(st)
        if i < 3:
            for b_i in range(3):
                scale, shift = norm_params(sts[b_i], gws[b_i][i], gbs[b_i][i],
                                           gmss[b_i][i])
                hs[b_i], zs[b_i] = _norm_act(Ss[b_i], scale, shift, disc)

    outs = []
    for b_i, (lW, lb, use_softplus) in enumerate(
            [(lin1_W, lin1_b, True), (lin2_W, lin2_b, False), (lin3_W, lin3_b, False)]):
        scale, shift = norm_params(sts[b_i], gws[b_i][3], gbs[b_i][3], gmss[b_i][3])
        pst = _norm_res_pool(Ss[b_i], scale, shift, x0s[b_i])
        pooled = jax.nn.relu(pst[0] / N)
        o = pooled @ lW.T + lb
        if use_softplus:
            o = jax.nn.softplus(o)
        outs.append(jax.nn.softmax(o) * jax.nn.relu(o))
    return tuple(outs)


# speedup vs baseline: 1.1321x; 1.1321x over previous
import functools

import jax
import jax.numpy as jnp
from jax import lax
from jax.experimental import pallas as pl
from jax.experimental.pallas import tpu as pltpu
from jax.experimental.pallas import tpu_sc as plsc

N = 10000
E = 320000
D = 128
OUT = 16
K = 4

NC = 2
NS = 16
NW = NC * NS
EPW = E // NW
CHUNK = 80
NCHUNK = EPW // CHUNK
ACC_ROWS = 10240

R = 400
NBLK = N // R

def _build_gather_sum():
    mesh = plsc.VectorSubcoreMesh(core_axis_name="c", subcore_axis_name="s")

    @functools.partial(
        pl.kernel,
        out_type=jax.ShapeDtypeStruct((NC, ACC_ROWS, D), jnp.float32),
        mesh=mesh,
        scratch_types=[
            pltpu.VMEM((EPW,), jnp.int32),
            pltpu.VMEM((NCHUNK, CHUNK), jnp.int32),
            pltpu.VMEM((2 * CHUNK, D), jnp.float32),
            pltpu.VMEM_SHARED((ACC_ROWS, D), jnp.float32),
            pltpu.SemaphoreType.DMA,
            pltpu.SemaphoreType.DMA,
            pltpu.SemaphoreType.DMA,
        ],
    )
    def gs(z_hbm, src_hbm, dst_hbm, out_hbm, sidx, didx, rows, acc, gsem, isem, ssem):
        c = lax.axis_index("c")
        s = lax.axis_index("s")
        wid = c * NS + s

        cp_s = pltpu.async_copy(src_hbm.at[wid], sidx, isem)
        cp_d = pltpu.async_copy(dst_hbm.at[wid], didx, isem)

        zeros = jnp.zeros((16,), jnp.float32)
        for i in range(16):
            for j in range(D // 16):
                rows[i, pl.ds(16 * j, 16)] = zeros

        @pl.loop(0, 640 // 16)
        def _zero(k):
            pltpu.sync_copy(rows.at[pl.ds(0, 16)], acc.at[pl.ds(s * 640 + k * 16, 16)])

        cp_s.wait()
        cp_d.wait()
        plsc.subcore_barrier()

        pltpu.async_copy(z_hbm.at[sidx.at[pl.ds(0, CHUNK)]], rows.at[pl.ds(0, CHUNK)], gsem)

        @pl.loop(0, NCHUNK)
        def _edges(k):
            b = (k % 2) * CHUNK
            pltpu.make_async_copy(
                z_hbm.at[sidx.at[pl.ds(k * CHUNK, CHUNK)]], rows.at[pl.ds(b, CHUNK)], gsem
            ).wait()

            @pl.when(k + 1 < NCHUNK)
            def _():
                nb = ((k + 1) % 2) * CHUNK

                @pl.when(k >= 1)
                def _():
                    pltpu.make_async_copy(
                        z_hbm.at[pl.ds(0, CHUNK)], rows.at[pl.ds(nb, CHUNK)], ssem
                    ).wait()

                pltpu.async_copy(
                    z_hbm.at[sidx.at[pl.ds((k + 1) * CHUNK, CHUNK)]],
                    rows.at[pl.ds(nb, CHUNK)], gsem
                )

            pltpu.async_copy(rows.at[pl.ds(b, CHUNK)], acc.at[didx.at[k]], ssem, add=True)

        pltpu.make_async_copy(z_hbm.at[pl.ds(0, 2 * CHUNK)], rows, ssem).wait()
        plsc.subcore_barrier()
        pltpu.sync_copy(acc.at[pl.ds(s * 640, 640)], out_hbm.at[c, pl.ds(s * 640, 640)])

    return gs


_gather_sum_cache = []


def _gather_sum(z, src, dst):
    if not _gather_sum_cache:
        _gather_sum_cache.append(_build_gather_sum())
    return _gather_sum_cache[0](z, src, dst)


def _build_degree():
    mesh = plsc.VectorSubcoreMesh(core_axis_name="c", subcore_axis_name="s")

    @functools.partial(
        pl.kernel,
        out_type=jax.ShapeDtypeStruct((NC, ACC_ROWS), jnp.float32),
        mesh=mesh,
        scratch_types=[
            pltpu.VMEM((NCHUNK, CHUNK), jnp.int32),
            pltpu.VMEM((CHUNK,), jnp.float32),
            pltpu.VMEM_SHARED((ACC_ROWS,), jnp.float32),
            pltpu.SemaphoreType.DMA,
        ],
    )
    def dg(src_hbm, out_hbm, sidx, ones, acc, isem):
        c = lax.axis_index("c")
        s = lax.axis_index("s")
        wid = c * NS + s

        cp = pltpu.async_copy(src_hbm.at[wid], sidx, isem)

        zeros16 = jnp.zeros((16,), jnp.float32)
        for j in range(CHUNK // 16):
            ones[pl.ds(16 * j, 16)] = zeros16

        @pl.loop(0, 640 // 16)
        def _zero(k):
            pltpu.sync_copy(ones.at[pl.ds(0, 16)], acc.at[pl.ds(s * 640 + k * 16, 16)])

        ones16 = jnp.full((16,), 1.0, jnp.float32)
        for j in range(CHUNK // 16):
            ones[pl.ds(16 * j, 16)] = ones16

        cp.wait()
        plsc.subcore_barrier()

        @pl.loop(0, NCHUNK)
        def _edges(k):
            pltpu.sync_copy(ones, acc.at[sidx.at[k]], add=True)

        plsc.subcore_barrier()
        pltpu.sync_copy(acc.at[pl.ds(s * 640, 640)], out_hbm.at[c, pl.ds(s * 640, 640)])

    return dg


_degree_cache = []


def _degree(src):
    if not _degree_cache:
        _degree_cache.append(_build_degree())
    return _degree_cache[0](src)


def _combine_stats_body(h_ref, g1_ref, g2_ref, g3_ref, dis_ref, V_ref, b_ref,
                        S_ref, st_ref, acc_ref):
    i = pl.program_id(0)
    d = dis_ref[...]
    u1 = d * (g1_ref[0] + g1_ref[1])
    u2 = d * (g2_ref[0] + g2_ref[1])
    u3 = d * (g3_ref[0] + g3_ref[1])
    S = jnp.dot(h_ref[...], V_ref[0], preferred_element_type=jnp.float32)
    S += jnp.dot(u1, V_ref[1], preferred_element_type=jnp.float32)
    S += jnp.dot(u2, V_ref[2], preferred_element_type=jnp.float32)
    S += jnp.dot(u3, V_ref[3], preferred_element_type=jnp.float32)
    S += b_ref[...]
    S_ref[...] = S

    @pl.when(i == 0)
    def _():
        acc_ref[...] = jnp.zeros_like(acc_ref)

    acc_ref[0:1, :] += jnp.sum(S, axis=0, keepdims=True)
    acc_ref[1:2, :] += jnp.sum(S * S, axis=0, keepdims=True)

    @pl.when(i == NBLK - 1)
    def _():
        st_ref[...] = acc_ref[...]


def _combine_stats(h, g1, g2, g3, dis, V, b):
    return pl.pallas_call(
        _combine_stats_body,
        grid=(NBLK,),
        in_specs=[
            pl.BlockSpec((R, D), lambda i: (i, 0)),
            pl.BlockSpec((2, R, D), lambda i: (0, i, 0)),
            pl.BlockSpec((2, R, D), lambda i: (0, i, 0)),
            pl.BlockSpec((2, R, D), lambda i: (0, i, 0)),
            pl.BlockSpec((R, 1), lambda i: (i, 0)),
            pl.BlockSpec((4, D, D), lambda i: (0, 0, 0)),
            pl.BlockSpec((1, D), lambda i: (0, 0)),
        ],
        out_specs=[
            pl.BlockSpec((R, D), lambda i: (i, 0)),
            pl.BlockSpec((8, 128), lambda i: (0, 0)),
        ],
        out_shape=[
            jax.ShapeDtypeStruct((N, D), jnp.float32),
            jax.ShapeDtypeStruct((8, 128), jnp.float32),
        ],
        scratch_shapes=[pltpu.VMEM((8, 128), jnp.float32)],
    )(h, g1, g2, g3, dis, V, b)


def _norm_act_body(S_ref, sc_ref, sh_ref, dis_ref, h_ref, z_ref):
    v = S_ref[...] * sc_ref[...] + sh_ref[...]
    h = jnp.where(v > 0, v, 0.1 * v)
    h_ref[...] = h
    z_ref[...] = dis_ref[...] * h


def _norm_act(S, scale, shift, dis):
    return pl.pallas_call(
        _norm_act_body,
        grid=(NBLK,),
        in_specs=[
            pl.BlockSpec((R, D), lambda i: (i, 0)),
            pl.BlockSpec((1, D), lambda i: (0, 0)),
            pl.BlockSpec((1, D), lambda i: (0, 0)),
            pl.BlockSpec((R, 1), lambda i: (i, 0)),
        ],
        out_specs=[
            pl.BlockSpec((R, D), lambda i: (i, 0)),
            pl.BlockSpec((R, D), lambda i: (i, 0)),
        ],
        out_shape=[
            jax.ShapeDtypeStruct((N, D), jnp.float32),
            jax.ShapeDtypeStruct((N, D), jnp.float32),
        ],
    )(S, scale, shift, dis)


def _norm_res_pool_body(S_ref, sc_ref, sh_ref, x0_ref, st_ref, acc_ref):
    i = pl.program_id(0)
    v = x0_ref[...] + S_ref[...] * sc_ref[...] + sh_ref[...]
    h = jnp.maximum(v, 0.0)

    @pl.when(i == 0)
    def _():
        acc_ref[...] = jnp.zeros_like(acc_ref)

    acc_ref[0:1, :] += jnp.sum(h, axis=0, keepdims=True)

    @pl.when(i == NBLK - 1)
    def _():
        st_ref[...] = acc_ref[...]


def _norm_res_pool(S, scale, shift, x0):
    return pl.pallas_call(
        _norm_res_pool_body,
        grid=(NBLK,),
        in_specs=[
            pl.BlockSpec((R, D), lambda i: (i, 0)),
            pl.BlockSpec((1, D), lambda i: (0, 0)),
            pl.BlockSpec((1, D), lambda i: (0, 0)),
            pl.BlockSpec((R, D), lambda i: (i, 0)),
        ],
        out_specs=pl.BlockSpec((8, 128), lambda i: (0, 0)),
        out_shape=jax.ShapeDtypeStruct((8, 128), jnp.float32),
        scratch_shapes=[pltpu.VMEM((8, 128), jnp.float32)],
    )(S, scale, shift, x0)


def _merge_body(g_ref, d_ref, t_ref, *, a):
    dd = d_ref[...]
    t_ref[...] = a * dd * dd * (g_ref[0] + g_ref[1])


def _merge_aux_body(g_ref, d_ref, aux_ref, t_ref, *, a, c):
    dd = d_ref[...]
    t_ref[...] = a * dd * dd * (g_ref[0] + g_ref[1]) + c * aux_ref[...]


def _merge_t(g, dis, a, aux=None, c=0.0):
    gspec = pl.BlockSpec((2, R, D), lambda i: (0, i, 0))
    dspec = pl.BlockSpec((R, 1), lambda i: (i, 0))
    xspec = pl.BlockSpec((R, D), lambda i: (i, 0))
    oshape = jax.ShapeDtypeStruct((N, D), jnp.float32)
    if aux is None:
        return pl.pallas_call(
            functools.partial(_merge_body, a=a),
            grid=(NBLK,), in_specs=[gspec, dspec], out_specs=xspec,
            out_shape=oshape,
        )(g, dis)
    return pl.pallas_call(
        functools.partial(_merge_aux_body, a=a, c=c),
        grid=(NBLK,), in_specs=[gspec, dspec, xspec], out_specs=xspec,
        out_shape=oshape,
    )(g, dis, aux)


def _scale_body(x_ref, d_ref, z_ref):
    z_ref[...] = d_ref[...] * x_ref[...]


def _scale_rows(x, dis):
    return pl.pallas_call(
        _scale_body,
        grid=(NBLK,),
        in_specs=[
            pl.BlockSpec((R, D), lambda i: (i, 0)),
            pl.BlockSpec((R, 1), lambda i: (i, 0)),
        ],
        out_specs=pl.BlockSpec((R, D), lambda i: (i, 0)),
        out_shape=jax.ShapeDtypeStruct((N, D), jnp.float32),
    )(x, dis)


def kernel(edge_index, feat, feat_1, feat_2,
           W1, b1, gn1_w, gn1_b, gn1_ms, lin1_W, lin1_b,
           W2, b2, gn2_w, gn2_b, gn2_ms, lin2_W, lin2_b,
           W3, b3, gn3_w, gn3_b, gn3_ms, lin3_W, lin3_b):
    src = edge_index[0].astype(jnp.int32)
    dst = edge_index[1].astype(jnp.int32)
    src3 = src.reshape(NW, EPW)
    dst3 = dst.reshape(NW, NCHUNK, CHUNK)
    dp = _degree(src.reshape(NW, NCHUNK, CHUNK))
    deg = dp[0, :N] + dp[1, :N]
    dis = jnp.where(deg > 0, 1.0 / jnp.sqrt(jnp.maximum(deg, 1e-12)), 0.0)
    disc = dis[:, None]

    def fold_W(W):
        return jnp.stack([W[0] - W[2], W[3] - W[1], -2.0 * W[2], -2.0 * W[3]])

    def norm_params(st, gw, gb, gms):
        mean = st[0] / N
        ex2 = st[1] / N
        var = ex2 - mean * mean * gms * (2.0 - gms)
        scale = gw / jnp.sqrt(var + 1e-5)
        shift = gb - scale * gms * mean
        return scale[None, :], shift[None, :]

    x0s = [feat, feat_1, feat_2]
    Ws = [W1, W2, W3]
    bs = [b1, b2, b3]
    gws = [gn1_w, gn2_w, gn3_w]
    gbs = [gn1_b, gn2_b, gn3_b]
    gmss = [gn1_ms, gn2_ms, gn3_ms]

    hs = list(x0s)
    zs = [_scale_rows(x, disc) for x in x0s]

    for i in range(4):
        g1s = [_gather_sum(z, src3, dst3) for z in zs]
        t1s = [_merge_t(g1, disc, -1.0) for g1 in g1s]
        g2s = [_gather_sum(t1, src3, dst3) for t1 in t1s]
        t2s = [_merge_t(g2, disc, -2.0, aux=z, c=-1.0)
               for g2, z in zip(g2s, zs)]
        g3s = [_gather_sum(t2, src3, dst3) for t2 in t2s]
        Ss, sts = [], []
        for b_i in range(3):
            S, st = _combine_stats(hs[b_i], g1s[b_i], g2s[b_i], g3s[b_i],
                                   disc, fold_W(Ws[b_i][i]), bs[b_i][i][None, :])
            Ss.append(S)
            sts.append(st)
        if i < 3:
            for b_i in range(3):
                scale, shift = norm_params(sts[b_i], gws[b_i][i], gbs[b_i][i],
                                           gmss[b_i][i])
                hs[b_i], zs[b_i] = _norm_act(Ss[b_i], scale, shift, disc)

    outs = []
    for b_i, (lW, lb, use_softplus) in enumerate(
            [(lin1_W, lin1_b, True), (lin2_W, lin2_b, False), (lin3_W, lin3_b, False)]):
        scale, shift = norm_params(sts[b_i], gws[b_i][3], gbs[b_i][3], gmss[b_i][3])
        pst = _norm_res_pool(Ss[b_i], scale, shift, x0s[b_i])
        pooled = jax.nn.relu(pst[0] / N)
        o = pooled @ lW.T + lb
        if use_softplus:
            o = jax.nn.softplus(o)
        outs.append(jax.nn.softmax(o) * jax.nn.relu(o))
    return tuple(outs)
